# baseline (speedup 1.0000x reference)
import functools

import jax
import jax.numpy as jnp
from jax import lax
from jax.experimental import pallas as pl
from jax.experimental.pallas import tpu as pltpu
from jax.experimental.pallas import tpu_sc as plsc

NC = 2
NS = 16
LANES = 16
NW = NC * NS
CH = 128


def _rows_per_tile(N):
    return ((N + NS * 8 - 1) // (NS * 8)) * 8


def _sc_segment_relu_sum(srcs, dsts, x, edge_weight):
    N, D = x.shape
    E = edge_weight.shape[0]
    CPT = srcs.shape[0] // (NW * CH)
    rpt = _rows_per_tile(N)
    Npad = rpt * NS
    ew_max = E - CH
    last = CPT - 1

    mesh = plsc.VectorSubcoreMesh(core_axis_name="c", subcore_axis_name="s")

    scratch = [
        pltpu.VMEM((CH,), jnp.int32),
        pltpu.VMEM((CH,), jnp.int32),
        pltpu.VMEM((CH,), jnp.int32),
        pltpu.VMEM((CH,), jnp.int32),
        pltpu.VMEM((CH,), jnp.int32),
        pltpu.VMEM((CH,), jnp.int32),
        pltpu.VMEM((CH, D), jnp.float32),
        pltpu.VMEM((CH, D), jnp.float32),
        pltpu.VMEM((CH, D), jnp.float32),
        pltpu.VMEM_SHARED((Npad, D), jnp.float32),
    ] + [pltpu.SemaphoreType.DMA] * 7

    @functools.partial(
        pl.kernel,
        out_type=jax.ShapeDtypeStruct((NC, Npad, D), jnp.float32),
        mesh=mesh,
        scratch_types=scratch,
    )
    def k(srcs_hbm, dsts_hbm, x_hbm, ew_hbm, out_hbm,
          srcv0, srcv1, dstv0, dstv1, sdx0, sdx1, rowsb, ewb0, ewb1, agg_sh,
          si0, si1, sg, se0, se1, ss0, ss1):
        cid = lax.axis_index("c")
        sid = lax.axis_index("s")
        wid = cid * NS + sid
        chunk_lo = wid * CPT
        srcv = (srcv0, srcv1)
        dstv = (dstv0, dstv1)
        sdx = (sdx0, sdx1)
        ewb = (ewb0, ewb1)
        sis = (si0, si1)
        ses = (se0, se1)
        sss = (ss0, ss1)

        zero16 = jnp.zeros((LANES,), jnp.float32)

        @pl.loop(0, CH)
        def _(r):
            for c0 in range(0, D, LANES):
                rowsb[r, pl.ds(c0, LANES)] = zero16

        row0 = sid * rpt
        zblocks = [CH] * (rpt // CH) + ([rpt % CH] if rpt % CH else [])
        off = 0
        for zb in zblocks:
            pltpu.sync_copy(rowsb.at[pl.ds(0, zb)],
                            agg_sh.at[pl.ds(row0 + off, zb)])
            off += zb
        plsc.subcore_barrier()

        def _idx_off(c):
            return pl.multiple_of((chunk_lo + c) * CH, 8)

        def _ew_off(c):
            return pl.multiple_of(jnp.minimum((chunk_lo + c) * CH, ew_max), 8)

        def issue_idx(c, b):
            off = _idx_off(c)
            pltpu.async_copy(srcs_hbm.at[pl.ds(off, CH)], srcv[b], sis[b])
            pltpu.async_copy(dsts_hbm.at[pl.ds(off, CH)], dstv[b], sis[b])

        def wait_idx(c, b):
            off = _idx_off(c)
            pltpu.make_async_copy(srcs_hbm.at[pl.ds(off, CH)], srcv[b], sis[b]).wait()
            pltpu.make_async_copy(dsts_hbm.at[pl.ds(off, CH)], dstv[b], sis[b]).wait()

        def issue_ew(c, b):
            pltpu.async_copy(ew_hbm.at[pl.ds(_ew_off(c), CH)], ewb[b], ses[b])

        def wait_ew(c, b):
            pltpu.make_async_copy(ew_hbm.at[pl.ds(_ew_off(c), CH)],
                                  ewb[b], ses[b]).wait()

        def issue_gather(b):
            pltpu.async_copy(x_hbm.at[srcv[b]], rowsb, sg)

        def wait_gather(b):
            pltpu.make_async_copy(x_hbm.at[srcv[b]], rowsb, sg).wait()

        def compute_into_ew(p):
            eb = ewb[p]

            @pl.loop(0, CH)
            def _(r):
                for c0 in range(0, D, LANES):
                    v = rowsb[r, pl.ds(c0, LANES)] + eb[r, pl.ds(c0, LANES)]
                    eb[r, pl.ds(c0, LANES)] = jnp.maximum(v, 0.0)

        def copy_scatter_idx(p):
            for j in range(0, CH, LANES):
                sdx[p][pl.ds(j, LANES)] = dstv[p][pl.ds(j, LANES)]

        def step(c, p, first=False):
            o = 1 - p
            wait_idx(jnp.minimum(c + 1, last), o)
            wait_gather(p)
            wait_ew(c, p)
            compute_into_ew(p)
            copy_scatter_idx(p)
            issue_gather(o)
            pltpu.async_copy(ewb[p], agg_sh.at[sdx[p]], sss[p], add=True)
            if not first:
                pltpu.make_async_copy(ewb[o], agg_sh.at[sdx[o]], sss[o]).wait()
                issue_ew(jnp.minimum(c + 1, last), o)
            issue_idx(jnp.minimum(c + 2, last), p)

        issue_idx(0, 0)
        issue_idx(1, 1)
        wait_idx(0, 0)
        issue_gather(0)
        issue_ew(0, 0)
        issue_ew(1, 1)
        step(0, 0, first=True)
        step(1, 1)

        @pl.loop(0, (CPT - 2) // 2)
        def _(kk):
            c0 = 2 + 2 * kk
            step(c0, 0)
            step(c0 + 1, 1)

        pltpu.make_async_copy(ewb[1], agg_sh.at[sdx[1]], sss[1]).wait()
        wait_idx(last, 1)
        wait_ew(last, 0)
        wait_gather(0)
        plsc.subcore_barrier()

        off = 0
        for zb in zblocks:
            r0 = row0 + off
            pltpu.sync_copy(agg_sh.at[pl.ds(r0, zb)], out_hbm.at[cid, pl.ds(r0, zb)])
            off += zb

    return k(srcs, dsts, x, edge_weight)[:, :N, :]


def _tc_linear(x, agg, Wt, b, eps):
    N, D = x.shape
    BN = 1000
    eps11 = jnp.reshape(eps, (1, 1)).astype(jnp.float32)
    b2d = jnp.reshape(b, (1, D))

    def body(eps_ref, x_ref, a0_ref, a1_ref, w_ref, b_ref, o_ref):
        rst = (1.0 + eps_ref[0, 0]) * x_ref[...] + a0_ref[...] + a1_ref[...]
        o_ref[...] = (jnp.dot(rst, w_ref[...], preferred_element_type=jnp.float32)
                      + b_ref[...])

    return pl.pallas_call(
        body,
        grid=(N // BN,),
        in_specs=[
            pl.BlockSpec(memory_space=pltpu.SMEM),
            pl.BlockSpec((BN, D), lambda i: (i, 0)),
            pl.BlockSpec((BN, D), lambda i: (i, 0)),
            pl.BlockSpec((BN, D), lambda i: (i, 0)),
            pl.BlockSpec((D, D), lambda i: (0, 0)),
            pl.BlockSpec((1, D), lambda i: (0, 0)),
        ],
        out_specs=pl.BlockSpec((BN, D), lambda i: (i, 0)),
        out_shape=jax.ShapeDtypeStruct((N, D), jnp.float32),
    )(eps11, x, agg[0], agg[1], Wt, b2d)


def _tc_lstm(xs, Wi0t, Wh0t, bb0, Wi1t, Wh1t, bb1):
    N, D = xs[0].shape
    H = D
    BN = 1000

    def body(x1_ref, x2_ref, x3_ref, x4_ref, wi0, wh0, b0, wi1, wh1, b1, o_ref):
        zeros = jnp.zeros((BN, H), jnp.float32)
        h0, c0, h1, c1, acc = zeros, zeros, zeros, zeros, zeros
        for x_ref in (x1_ref, x2_ref, x3_ref, x4_ref):
            xt = x_ref[...]
            g = (jnp.dot(xt, wi0[...], preferred_element_type=jnp.float32)
                 + jnp.dot(h0, wh0[...], preferred_element_type=jnp.float32)
                 + b0[...])
            i = jax.nn.sigmoid(g[:, 0 * H:1 * H])
            f = jax.nn.sigmoid(g[:, 1 * H:2 * H])
            gg = jnp.tanh(g[:, 2 * H:3 * H])
            o = jax.nn.sigmoid(g[:, 3 * H:4 * H])
            c0 = f * c0 + i * gg
            h0 = o * jnp.tanh(c0)
            g = (jnp.dot(h0, wi1[...], preferred_element_type=jnp.float32)
                 + jnp.dot(h1, wh1[...], preferred_element_type=jnp.float32)
                 + b1[...])
            i = jax.nn.sigmoid(g[:, 0 * H:1 * H])
            f = jax.nn.sigmoid(g[:, 1 * H:2 * H])
            gg = jnp.tanh(g[:, 2 * H:3 * H])
            o = jax.nn.sigmoid(g[:, 3 * H:4 * H])
            c1 = f * c1 + i * gg
            h1 = o * jnp.tanh(c1)
            acc = acc + h1
        o_ref[...] = acc * 0.25

    wspec = pl.BlockSpec((D, 4 * H), lambda i: (0, 0))
    bspec = pl.BlockSpec((1, 4 * H), lambda i: (0, 0))
    xspec = pl.BlockSpec((BN, D), lambda i: (i, 0))
    return pl.pallas_call(
        body,
        grid=(N // BN,),
        in_specs=[xspec, xspec, xspec, xspec,
                  wspec, wspec, bspec, wspec, wspec, bspec],
        out_specs=pl.BlockSpec((BN, H), lambda i: (i, 0)),
        out_shape=jax.ShapeDtypeStruct((N, H), jnp.float32),
    )(*xs, Wi0t, Wh0t, bb0, Wi1t, Wh1t, bb1)


def kernel(edge_index, x, edge_weight, W1, b1, eps1, W2, b2, eps2, W3, b3,
           eps3, W4, b4, eps4, W_ih0, W_hh0, b_ih0, b_hh0, W_ih1, W_hh1,
           b_ih1, b_hh1):
    layers = ((W1, b1, eps1), (W2, b2, eps2), (W3, b3, eps3), (W4, b4, eps4))
    N = x.shape[0]
    E = edge_index.shape[1]
    n_real_chunks = -(-E // CH)
    cpt = -(-n_real_chunks // NW)
    if cpt % 2:
        cpt += 1
    Epad = NW * cpt * CH
    npad = _rows_per_tile(N) * NS
    srcs = jnp.concatenate([edge_index[0], jnp.zeros((Epad - E,), jnp.int32)])
    dsts = jnp.concatenate(
        [edge_index[1], jnp.full((Epad - E,), npad - 1, jnp.int32)])
    xs = []
    h = x
    for W, b, eps in layers:
        agg = _sc_segment_relu_sum(srcs, dsts, h, edge_weight)
        h = _tc_linear(h, agg, W.T, b, eps)
        xs.append(h)
    bb0 = jnp.reshape(b_ih0 + b_hh0, (1, -1))
    bb1 = jnp.reshape(b_ih1 + b_hh1, (1, -1))
    return _tc_lstm(xs, W_ih0.T, W_hh0.T, bb0, W_ih1.T, W_hh1.T, bb1)


# speedup vs baseline: 1.0006x; 1.0006x over previous
import functools

import jax
import jax.numpy as jnp
from jax import lax
from jax.experimental import pallas as pl
from jax.experimental.pallas import tpu as pltpu
from jax.experimental.pallas import tpu_sc as plsc

NC = 2
NS = 16
LANES = 16
NW = NC * NS
CH = 128


def _rows_per_tile(N):
    return ((N + NS * 8 - 1) // (NS * 8)) * 8


def _sc_segment_relu_sum(srcs, dsts, x, edge_weight):
    N, D = x.shape
    E = edge_weight.shape[0]
    CPT = srcs.shape[0] // (NW * CH)
    rpt = _rows_per_tile(N)
    Npad = rpt * NS
    ew_max = E - CH
    last = CPT - 1

    mesh = plsc.VectorSubcoreMesh(core_axis_name="c", subcore_axis_name="s")

    scratch = [
        pltpu.VMEM((CH,), jnp.int32),
        pltpu.VMEM((CH,), jnp.int32),
        pltpu.VMEM((CH,), jnp.int32),
        pltpu.VMEM((CH,), jnp.int32),
        pltpu.VMEM((CH,), jnp.int32),
        pltpu.VMEM((CH,), jnp.int32),
        pltpu.VMEM((CH, D), jnp.float32),
        pltpu.VMEM((CH, D), jnp.float32),
        pltpu.VMEM((CH, D), jnp.float32),
        pltpu.VMEM_SHARED((Npad, D), jnp.float32),
    ] + [pltpu.SemaphoreType.DMA] * 7

    @functools.partial(
        pl.kernel,
        out_type=jax.ShapeDtypeStruct((NC, Npad, D), jnp.float32),
        mesh=mesh,
        scratch_types=scratch,
    )
    def k(srcs_hbm, dsts_hbm, x_hbm, ew_hbm, out_hbm,
          srcv0, srcv1, dstv0, dstv1, sdx0, sdx1, rowsb, ewb0, ewb1, agg_sh,
          si0, si1, sg, se0, se1, ss0, ss1):
        cid = lax.axis_index("c")
        sid = lax.axis_index("s")
        wid = cid * NS + sid
        chunk_lo = wid * CPT
        srcv = (srcv0, srcv1)
        dstv = (dstv0, dstv1)
        sdx = (sdx0, sdx1)
        ewb = (ewb0, ewb1)
        sis = (si0, si1)
        ses = (se0, se1)
        sss = (ss0, ss1)

        zero16 = jnp.zeros((LANES,), jnp.float32)

        @pl.loop(0, CH)
        def _(r):
            for c0 in range(0, D, LANES):
                rowsb[r, pl.ds(c0, LANES)] = zero16

        row0 = sid * rpt
        zblocks = [CH] * (rpt // CH) + ([rpt % CH] if rpt % CH else [])
        off = 0
        for zb in zblocks:
            pltpu.sync_copy(rowsb.at[pl.ds(0, zb)],
                            agg_sh.at[pl.ds(row0 + off, zb)])
            off += zb
        plsc.subcore_barrier()

        def _idx_off(c):
            return pl.multiple_of((chunk_lo + c) * CH, 8)

        def _ew_off(c):
            return pl.multiple_of(jnp.minimum((chunk_lo + c) * CH, ew_max), 8)

        def issue_idx(c, b):
            off = _idx_off(c)
            pltpu.async_copy(srcs_hbm.at[pl.ds(off, CH)], srcv[b], sis[b])
            pltpu.async_copy(dsts_hbm.at[pl.ds(off, CH)], dstv[b], sis[b])

        def wait_idx(c, b):
            off = _idx_off(c)
            pltpu.make_async_copy(srcs_hbm.at[pl.ds(off, CH)], srcv[b], sis[b]).wait()
            pltpu.make_async_copy(dsts_hbm.at[pl.ds(off, CH)], dstv[b], sis[b]).wait()

        def issue_ew(c, b):
            pltpu.async_copy(ew_hbm.at[pl.ds(_ew_off(c), CH)], ewb[b], ses[b])

        def wait_ew(c, b):
            pltpu.make_async_copy(ew_hbm.at[pl.ds(_ew_off(c), CH)],
                                  ewb[b], ses[b]).wait()

        def issue_gather(b):
            pltpu.async_copy(x_hbm.at[srcv[b]], rowsb, sg)

        def wait_gather(b):
            pltpu.make_async_copy(x_hbm.at[srcv[b]], rowsb, sg).wait()

        def compute_into_ew(p):
            eb = ewb[p]

            @pl.loop(0, CH)
            def _(r):
                for c0 in range(0, D, LANES):
                    v = rowsb[r, pl.ds(c0, LANES)] + eb[r, pl.ds(c0, LANES)]
                    eb[r, pl.ds(c0, LANES)] = jnp.maximum(v, 0.0)

        def copy_scatter_idx(p):
            for j in range(0, CH, LANES):
                sdx[p][pl.ds(j, LANES)] = dstv[p][pl.ds(j, LANES)]

        def step(c, p, first=False):
            o = 1 - p
            wait_idx(jnp.minimum(c + 1, last), o)
            wait_gather(p)
            wait_ew(c, p)
            compute_into_ew(p)
            copy_scatter_idx(p)
            issue_gather(o)
            pltpu.async_copy(ewb[p], agg_sh.at[sdx[p]], sss[p], add=True)
            if not first:
                pltpu.make_async_copy(ewb[o], agg_sh.at[sdx[o]], sss[o]).wait()
                issue_ew(jnp.minimum(c + 1, last), o)
            issue_idx(jnp.minimum(c + 2, last), p)

        issue_idx(0, 0)
        issue_idx(1, 1)
        wait_idx(0, 0)
        issue_gather(0)
        issue_ew(0, 0)
        issue_ew(1, 1)
        step(0, 0, first=True)
        step(1, 1)

        @pl.loop(0, (CPT - 2) // 2)
        def _(kk):
            c0 = 2 + 2 * kk
            step(c0, 0)
            step(c0 + 1, 1)

        pltpu.make_async_copy(ewb[1], agg_sh.at[sdx[1]], sss[1]).wait()
        wait_idx(last, 1)
        wait_ew(last, 0)
        wait_gather(0)
        plsc.subcore_barrier()

        off = 0
        for zb in zblocks:
            r0 = row0 + off
            pltpu.sync_copy(agg_sh.at[pl.ds(r0, zb)], out_hbm.at[cid, pl.ds(r0, zb)])
            off += zb

    return k(srcs, dsts, x, edge_weight)[:, :N, :]


def _tc_linear(x, agg, Wt, b, eps):
    N, D = x.shape
    BN = 1000
    eps11 = jnp.reshape(eps, (1, 1)).astype(jnp.float32)
    b2d = jnp.reshape(b, (1, D))

    def body(eps_ref, x_ref, a0_ref, a1_ref, w_ref, b_ref, o_ref):
        rst = (1.0 + eps_ref[0, 0]) * x_ref[...] + a0_ref[...] + a1_ref[...]
        o_ref[...] = (jnp.dot(rst, w_ref[...], preferred_element_type=jnp.float32)
                      + b_ref[...])

    return pl.pallas_call(
        body,
        grid=(N // BN,),
        in_specs=[
            pl.BlockSpec(memory_space=pltpu.SMEM),
            pl.BlockSpec((BN, D), lambda i: (i, 0)),
            pl.BlockSpec((BN, D), lambda i: (i, 0)),
            pl.BlockSpec((BN, D), lambda i: (i, 0)),
            pl.BlockSpec((D, D), lambda i: (0, 0)),
            pl.BlockSpec((1, D), lambda i: (0, 0)),
        ],
        out_specs=pl.BlockSpec((BN, D), lambda i: (i, 0)),
        out_shape=jax.ShapeDtypeStruct((N, D), jnp.float32),
    )(eps11, x, agg[0], agg[1], Wt, b2d)


def _tc_lstm(xs, Wi0t, Wh0t, bb0, Wi1t, Wh1t, bb1):
    N, D = xs[0].shape
    H = D
    BN = 1000

    def body(x1_ref, x2_ref, x3_ref, x4_ref, wi0, wh0, b0, wi1, wh1, b1, o_ref):
        zeros = jnp.zeros((BN, H), jnp.float32)
        h0, c0, h1, c1, acc = zeros, zeros, zeros, zeros, zeros
        for x_ref in (x1_ref, x2_ref, x3_ref, x4_ref):
            xt = x_ref[...]
            g = (jnp.dot(xt, wi0[...], preferred_element_type=jnp.float32)
                 + jnp.dot(h0, wh0[...], preferred_element_type=jnp.float32)
                 + b0[...])
            i = jax.nn.sigmoid(g[:, 0 * H:1 * H])
            f = jax.nn.sigmoid(g[:, 1 * H:2 * H])
            gg = jnp.tanh(g[:, 2 * H:3 * H])
            o = jax.nn.sigmoid(g[:, 3 * H:4 * H])
            c0 = f * c0 + i * gg
            h0 = o * jnp.tanh(c0)
            g = (jnp.dot(h0, wi1[...], preferred_element_type=jnp.float32)
                 + jnp.dot(h1, wh1[...], preferred_element_type=jnp.float32)
                 + b1[...])
            i = jax.nn.sigmoid(g[:, 0 * H:1 * H])
            f = jax.nn.sigmoid(g[:, 1 * H:2 * H])
            gg = jnp.tanh(g[:, 2 * H:3 * H])
            o = jax.nn.sigmoid(g[:, 3 * H:4 * H])
            c1 = f * c1 + i * gg
            h1 = o * jnp.tanh(c1)
            acc = acc + h1
        o_ref[...] = acc * 0.25

    wspec = pl.BlockSpec((D, 4 * H), lambda i: (0, 0))
    bspec = pl.BlockSpec((1, 4 * H), lambda i: (0, 0))
    xspec = pl.BlockSpec((BN, D), lambda i: (i, 0))
    return pl.pallas_call(
        body,
        grid=(N // BN,),
        in_specs=[xspec, xspec, xspec, xspec,
                  wspec, wspec, bspec, wspec, wspec, bspec],
        out_specs=pl.BlockSpec((BN, H), lambda i: (i, 0)),
        out_shape=jax.ShapeDtypeStruct((N, H), jnp.float32),
    )(*xs, Wi0t, Wh0t, bb0, Wi1t, Wh1t, bb1)


def kernel(edge_index, x, edge_weight, W1, b1, eps1, W2, b2, eps2, W3, b3,
           eps3, W4, b4, eps4, W_ih0, W_hh0, b_ih0, b_hh0, W_ih1, W_hh1,
           b_ih1, b_hh1):
    layers = ((W1, b1, eps1), (W2, b2, eps2), (W3, b3, eps3), (W4, b4, eps4))
    N = x.shape[0]
    E = edge_index.shape[1]
    n_real_chunks = -(-E // CH)
    cpt = -(-n_real_chunks // NW)
    if cpt % 2:
        cpt += 1
    Epad = NW * cpt * CH
    npad = _rows_per_tile(N) * NS
    srcs = jnp.concatenate([edge_index[0], jnp.zeros((Epad - E,), jnp.int32)])
    pad_dst = N + jnp.arange(Epad - E, dtype=jnp.int32) % (npad - N)
    dsts = jnp.concatenate([edge_index[1], pad_dst])
    xs = []
    h = x
    for W, b, eps in layers:
        agg = _sc_segment_relu_sum(srcs, dsts, h, edge_weight)
        h = _tc_linear(h, agg, W.T, b, eps)
        xs.append(h)
    bb0 = jnp.reshape(b_ih0 + b_hh0, (1, -1))
    bb1 = jnp.reshape(b_ih1 + b_hh1, (1, -1))
    return _tc_lstm(xs, W_ih0.T, W_hh0.T, bb0, W_ih1.T, W_hh1.T, bb1)


# speedup vs baseline: 1.5772x; 1.5762x over previous
import functools

import jax
import jax.numpy as jnp
from jax import lax
from jax.experimental import pallas as pl
from jax.experimental.pallas import tpu as pltpu
from jax.experimental.pallas import tpu_sc as plsc

NC = 2
NS = 16
LANES = 16
NW = NC * NS


def _sc_segment_relu_sum(src_arr, dst_arr, x, edge_weight):
    N, D = x.shape
    E = src_arr.shape[0]
    per_w = E // NW
    CH = 128
    n_chunks = per_w // CH
    tail = per_w - n_chunks * CH
    rows_per_tile = -(-N // (NS * CH)) * CH
    Npad = rows_per_tile * NS
    full, rem = divmod(rows_per_tile, CH)

    mesh = plsc.VectorSubcoreMesh(core_axis_name="c", subcore_axis_name="s")

    scratch = [
        pltpu.VMEM((CH,), jnp.int32),
        pltpu.VMEM((CH,), jnp.int32),
        pltpu.VMEM((CH, D), jnp.float32),
        pltpu.VMEM((CH, D), jnp.float32),
        pltpu.VMEM_SHARED((Npad, D), jnp.float32),
        pltpu.SemaphoreType.DMA,
    ]
    if tail:
        scratch += [
            pltpu.VMEM((tail,), jnp.int32),
            pltpu.VMEM((tail,), jnp.int32),
            pltpu.VMEM((tail, D), jnp.float32),
            pltpu.VMEM((tail, D), jnp.float32),
        ]

    @functools.partial(
        pl.kernel,
        out_type=jax.ShapeDtypeStruct((NC, Npad, D), jnp.float32),
        mesh=mesh,
        scratch_types=scratch,
    )
    def k(src_hbm, dst_hbm, x_hbm, ew_hbm, out_hbm, src_v, dst_v, rows_v, ew_v,
          agg_sh, sem, *tail_bufs):
        cid = lax.axis_index("c")
        sid = lax.axis_index("s")
        wid = cid * NS + sid

        zero16 = jnp.zeros((LANES,), jnp.float32)

        @pl.loop(0, CH)
        def _(r):
            for c0 in range(0, D, LANES):
                rows_v[r, pl.ds(c0, LANES)] = zero16

        row0 = sid * rows_per_tile
        for kb in range(full):
            pltpu.sync_copy(rows_v, agg_sh.at[pl.ds(row0 + kb * CH, CH)])
        if rem:
            pltpu.sync_copy(rows_v.at[pl.ds(0, rem)],
                            agg_sh.at[pl.ds(row0 + full * CH, rem)])
        plsc.subcore_barrier()

        ebase = wid * per_w

        @pl.loop(0, n_chunks)
        def _(ci):
            b = pl.multiple_of(ebase + ci * CH, 8)
            pltpu.sync_copy(src_hbm.at[pl.ds(b, CH)], src_v)
            pltpu.sync_copy(dst_hbm.at[pl.ds(b, CH)], dst_v)
            g = pltpu.async_copy(x_hbm.at[src_v], rows_v, sem)
            pltpu.sync_copy(ew_hbm.at[pl.ds(b, CH)], ew_v)
            g.wait()

            @pl.loop(0, CH)
            def _(r):
                for c0 in range(0, D, LANES):
                    v = rows_v[r, pl.ds(c0, LANES)] + ew_v[r, pl.ds(c0, LANES)]
                    rows_v[r, pl.ds(c0, LANES)] = jnp.maximum(v, 0.0)

            pltpu.sync_copy(rows_v, agg_sh.at[dst_v], add=True)

        if tail:
            src_t, dst_t, rows_t, ew_t = tail_bufs
            b = pl.multiple_of(ebase + n_chunks * CH, 8)
            pltpu.sync_copy(src_hbm.at[pl.ds(b, tail)], src_t)
            pltpu.sync_copy(dst_hbm.at[pl.ds(b, tail)], dst_t)
            g = pltpu.async_copy(x_hbm.at[src_t], rows_t, sem)
            pltpu.sync_copy(ew_hbm.at[pl.ds(b, tail)], ew_t)
            g.wait()

            @pl.loop(0, tail)
            def _(r):
                for c0 in range(0, D, LANES):
                    v = rows_t[r, pl.ds(c0, LANES)] + ew_t[r, pl.ds(c0, LANES)]
                    rows_t[r, pl.ds(c0, LANES)] = jnp.maximum(v, 0.0)

            pltpu.sync_copy(rows_t, agg_sh.at[dst_t], add=True)

        plsc.subcore_barrier()

        for kb in range(full):
            r0 = row0 + kb * CH
            pltpu.sync_copy(agg_sh.at[pl.ds(r0, CH)], out_hbm.at[cid, pl.ds(r0, CH)])
        if rem:
            r0 = row0 + full * CH
            pltpu.sync_copy(agg_sh.at[pl.ds(r0, rem)],
                            out_hbm.at[cid, pl.ds(r0, rem)])

    return k(src_arr, dst_arr, x, edge_weight)[:, :N, :]


def _tc_linear(x, agg, Wt, b, eps):
    N, D = x.shape
    BN = 1000
    eps11 = jnp.reshape(eps, (1, 1)).astype(jnp.float32)
    b2d = jnp.reshape(b, (1, D))

    def body(eps_ref, x_ref, a0_ref, a1_ref, w_ref, b_ref, o_ref):
        rst = (1.0 + eps_ref[0, 0]) * x_ref[...] + a0_ref[...] + a1_ref[...]
        o_ref[...] = (jnp.dot(rst, w_ref[...], preferred_element_type=jnp.float32)
                      + b_ref[...])

    return pl.pallas_call(
        body,
        grid=(N // BN,),
        in_specs=[
            pl.BlockSpec(memory_space=pltpu.SMEM),
            pl.BlockSpec((BN, D), lambda i: (i, 0)),
            pl.BlockSpec((BN, D), lambda i: (i, 0)),
            pl.BlockSpec((BN, D), lambda i: (i, 0)),
            pl.BlockSpec((D, D), lambda i: (0, 0)),
            pl.BlockSpec((1, D), lambda i: (0, 0)),
        ],
        out_specs=pl.BlockSpec((BN, D), lambda i: (i, 0)),
        out_shape=jax.ShapeDtypeStruct((N, D), jnp.float32),
    )(eps11, x, agg[0], agg[1], Wt, b2d)


def _tc_lstm(xs, Wi0t, Wh0t, bb0, Wi1t, Wh1t, bb1):
    N, D = xs[0].shape
    H = D
    BN = 1000

    def body(x1_ref, x2_ref, x3_ref, x4_ref, wi0, wh0, b0, wi1, wh1, b1, o_ref):
        zeros = jnp.zeros((BN, H), jnp.float32)
        h0, c0, h1, c1, acc = zeros, zeros, zeros, zeros, zeros
        for x_ref in (x1_ref, x2_ref, x3_ref, x4_ref):
            xt = x_ref[...]
            g = (jnp.dot(xt, wi0[...], preferred_element_type=jnp.float32)
                 + jnp.dot(h0, wh0[...], preferred_element_type=jnp.float32)
                 + b0[...])
            i = jax.nn.sigmoid(g[:, 0 * H:1 * H])
            f = jax.nn.sigmoid(g[:, 1 * H:2 * H])
            gg = jnp.tanh(g[:, 2 * H:3 * H])
            o = jax.nn.sigmoid(g[:, 3 * H:4 * H])
            c0 = f * c0 + i * gg
            h0 = o * jnp.tanh(c0)
            g = (jnp.dot(h0, wi1[...], preferred_element_type=jnp.float32)
                 + jnp.dot(h1, wh1[...], preferred_element_type=jnp.float32)
                 + b1[...])
            i = jax.nn.sigmoid(g[:, 0 * H:1 * H])
            f = jax.nn.sigmoid(g[:, 1 * H:2 * H])
            gg = jnp.tanh(g[:, 2 * H:3 * H])
            o = jax.nn.sigmoid(g[:, 3 * H:4 * H])
            c1 = f * c1 + i * gg
            h1 = o * jnp.tanh(c1)
            acc = acc + h1
        o_ref[...] = acc * 0.25

    wspec = pl.BlockSpec((D, 4 * H), lambda i: (0, 0))
    bspec = pl.BlockSpec((1, 4 * H), lambda i: (0, 0))
    xspec = pl.BlockSpec((BN, D), lambda i: (i, 0))
    return pl.pallas_call(
        body,
        grid=(N // BN,),
        in_specs=[xspec, xspec, xspec, xspec,
                  wspec, wspec, bspec, wspec, wspec, bspec],
        out_specs=pl.BlockSpec((BN, H), lambda i: (i, 0)),
        out_shape=jax.ShapeDtypeStruct((N, H), jnp.float32),
    )(*xs, Wi0t, Wh0t, bb0, Wi1t, Wh1t, bb1)


def kernel(edge_index, x, edge_weight, W1, b1, eps1, W2, b2, eps2, W3, b3,
           eps3, W4, b4, eps4, W_ih0, W_hh0, b_ih0, b_hh0, W_ih1, W_hh1,
           b_ih1, b_hh1):
    layers = ((W1, b1, eps1), (W2, b2, eps2), (W3, b3, eps3), (W4, b4, eps4))
    src_arr = edge_index[0]
    dst_arr = edge_index[1]
    xs = []
    h = x
    for W, b, eps in layers:
        agg = _sc_segment_relu_sum(src_arr, dst_arr, h, edge_weight)
        h = _tc_linear(h, agg, W.T, b, eps)
        xs.append(h)
    bb0 = jnp.reshape(b_ih0 + b_hh0, (1, -1))
    bb1 = jnp.reshape(b_ih1 + b_hh1, (1, -1))
    return _tc_lstm(xs, W_ih0.T, W_hh0.T, bb0, W_ih1.T, W_hh1.T, bb1)


# speedup vs baseline: 1.5798x; 1.0016x over previous
import functools

import jax
import jax.numpy as jnp
from jax import lax
from jax.experimental import pallas as pl
from jax.experimental.pallas import tpu as pltpu
from jax.experimental.pallas import tpu_sc as plsc

NC = 2
NS = 16
LANES = 16
NW = NC * NS


def _dot_bf16(a, w):
    return jnp.dot(a.astype(jnp.bfloat16), w.astype(jnp.bfloat16),
                   preferred_element_type=jnp.float32)


def _sc_segment_relu_sum(src_arr, dst_arr, x, edge_weight):
    N, D = x.shape
    E = src_arr.shape[0]
    per_w = E // NW
    CH = 128
    n_chunks = per_w // CH
    tail = per_w - n_chunks * CH
    rows_per_tile = -(-N // (NS * CH)) * CH
    Npad = rows_per_tile * NS
    full, rem = divmod(rows_per_tile, CH)

    mesh = plsc.VectorSubcoreMesh(core_axis_name="c", subcore_axis_name="s")

    scratch = [
        pltpu.VMEM((CH,), jnp.int32),
        pltpu.VMEM((CH,), jnp.int32),
        pltpu.VMEM((CH, D), jnp.float32),
        pltpu.VMEM((CH, D), jnp.float32),
        pltpu.VMEM_SHARED((Npad, D), jnp.float32),
        pltpu.SemaphoreType.DMA,
    ]
    if tail:
        scratch += [
            pltpu.VMEM((tail,), jnp.int32),
            pltpu.VMEM((tail,), jnp.int32),
            pltpu.VMEM((tail, D), jnp.float32),
            pltpu.VMEM((tail, D), jnp.float32),
        ]

    @functools.partial(
        pl.kernel,
        out_type=jax.ShapeDtypeStruct((NC, Npad, D), jnp.float32),
        mesh=mesh,
        scratch_types=scratch,
    )
    def k(src_hbm, dst_hbm, x_hbm, ew_hbm, out_hbm, src_v, dst_v, rows_v, ew_v,
          agg_sh, sem, *tail_bufs):
        cid = lax.axis_index("c")
        sid = lax.axis_index("s")
        wid = cid * NS + sid

        zero16 = jnp.zeros((LANES,), jnp.float32)

        @pl.loop(0, CH)
        def _(r):
            for c0 in range(0, D, LANES):
                rows_v[r, pl.ds(c0, LANES)] = zero16

        row0 = sid * rows_per_tile
        for kb in range(full):
            pltpu.sync_copy(rows_v, agg_sh.at[pl.ds(row0 + kb * CH, CH)])
        if rem:
            pltpu.sync_copy(rows_v.at[pl.ds(0, rem)],
                            agg_sh.at[pl.ds(row0 + full * CH, rem)])
        plsc.subcore_barrier()

        ebase = wid * per_w

        @pl.loop(0, n_chunks)
        def _(ci):
            b = pl.multiple_of(ebase + ci * CH, 8)
            pltpu.sync_copy(src_hbm.at[pl.ds(b, CH)], src_v)
            pltpu.sync_copy(dst_hbm.at[pl.ds(b, CH)], dst_v)
            g = pltpu.async_copy(x_hbm.at[src_v], rows_v, sem)
            pltpu.sync_copy(ew_hbm.at[pl.ds(b, CH)], ew_v)
            g.wait()

            @pl.loop(0, CH)
            def _(r):
                for c0 in range(0, D, LANES):
                    v = rows_v[r, pl.ds(c0, LANES)] + ew_v[r, pl.ds(c0, LANES)]
                    rows_v[r, pl.ds(c0, LANES)] = jnp.maximum(v, 0.0)

            pltpu.sync_copy(rows_v, agg_sh.at[dst_v], add=True)

        if tail:
            src_t, dst_t, rows_t, ew_t = tail_bufs
            b = pl.multiple_of(ebase + n_chunks * CH, 8)
            pltpu.sync_copy(src_hbm.at[pl.ds(b, tail)], src_t)
            pltpu.sync_copy(dst_hbm.at[pl.ds(b, tail)], dst_t)
            g = pltpu.async_copy(x_hbm.at[src_t], rows_t, sem)
            pltpu.sync_copy(ew_hbm.at[pl.ds(b, tail)], ew_t)
            g.wait()

            @pl.loop(0, tail)
            def _(r):
                for c0 in range(0, D, LANES):
                    v = rows_t[r, pl.ds(c0, LANES)] + ew_t[r, pl.ds(c0, LANES)]
                    rows_t[r, pl.ds(c0, LANES)] = jnp.maximum(v, 0.0)

            pltpu.sync_copy(rows_t, agg_sh.at[dst_t], add=True)

        plsc.subcore_barrier()

        for kb in range(full):
            r0 = row0 + kb * CH
            pltpu.sync_copy(agg_sh.at[pl.ds(r0, CH)], out_hbm.at[cid, pl.ds(r0, CH)])
        if rem:
            r0 = row0 + full * CH
            pltpu.sync_copy(agg_sh.at[pl.ds(r0, rem)],
                            out_hbm.at[cid, pl.ds(r0, rem)])

    return k(src_arr, dst_arr, x, edge_weight)[:, :N, :]


def _tc_linear(x, agg, Wt, b, eps):
    N, D = x.shape
    BN = 1000
    eps11 = jnp.reshape(eps, (1, 1)).astype(jnp.float32)
    b2d = jnp.reshape(b, (1, D))

    def body(eps_ref, x_ref, a0_ref, a1_ref, w_ref, b_ref, o_ref):
        rst = (1.0 + eps_ref[0, 0]) * x_ref[...] + a0_ref[...] + a1_ref[...]
        o_ref[...] = (_dot_bf16(rst, w_ref[...])
                      + b_ref[...])

    return pl.pallas_call(
        body,
        grid=(N // BN,),
        in_specs=[
            pl.BlockSpec(memory_space=pltpu.SMEM),
            pl.BlockSpec((BN, D), lambda i: (i, 0)),
            pl.BlockSpec((BN, D), lambda i: (i, 0)),
            pl.BlockSpec((BN, D), lambda i: (i, 0)),
            pl.BlockSpec((D, D), lambda i: (0, 0)),
            pl.BlockSpec((1, D), lambda i: (0, 0)),
        ],
        out_specs=pl.BlockSpec((BN, D), lambda i: (i, 0)),
        out_shape=jax.ShapeDtypeStruct((N, D), jnp.float32),
    )(eps11, x, agg[0], agg[1], Wt, b2d)


def _tc_lstm(xs, Wi0t, Wh0t, bb0, Wi1t, Wh1t, bb1):
    N, D = xs[0].shape
    H = D
    BN = 1000

    def body(x1_ref, x2_ref, x3_ref, x4_ref, wi0, wh0, b0, wi1, wh1, b1, o_ref):
        zeros = jnp.zeros((BN, H), jnp.float32)
        h0, c0, h1, c1, acc = zeros, zeros, zeros, zeros, zeros
        for x_ref in (x1_ref, x2_ref, x3_ref, x4_ref):
            xt = x_ref[...]
            g = (_dot_bf16(xt, wi0[...])
                 + _dot_bf16(h0, wh0[...])
                 + b0[...])
            i = jax.nn.sigmoid(g[:, 0 * H:1 * H])
            f = jax.nn.sigmoid(g[:, 1 * H:2 * H])
            gg = jnp.tanh(g[:, 2 * H:3 * H])
            o = jax.nn.sigmoid(g[:, 3 * H:4 * H])
            c0 = f * c0 + i * gg
            h0 = o * jnp.tanh(c0)
            g = (_dot_bf16(h0, wi1[...])
                 + _dot_bf16(h1, wh1[...])
                 + b1[...])
            i = jax.nn.sigmoid(g[:, 0 * H:1 * H])
            f = jax.nn.sigmoid(g[:, 1 * H:2 * H])
            gg = jnp.tanh(g[:, 2 * H:3 * H])
            o = jax.nn.sigmoid(g[:, 3 * H:4 * H])
            c1 = f * c1 + i * gg
            h1 = o * jnp.tanh(c1)
            acc = acc + h1
        o_ref[...] = acc * 0.25

    wspec = pl.BlockSpec((D, 4 * H), lambda i: (0, 0))
    bspec = pl.BlockSpec((1, 4 * H), lambda i: (0, 0))
    xspec = pl.BlockSpec((BN, D), lambda i: (i, 0))
    return pl.pallas_call(
        body,
        grid=(N // BN,),
        in_specs=[xspec, xspec, xspec, xspec,
                  wspec, wspec, bspec, wspec, wspec, bspec],
        out_specs=pl.BlockSpec((BN, H), lambda i: (i, 0)),
        out_shape=jax.ShapeDtypeStruct((N, H), jnp.float32),
    )(*xs, Wi0t, Wh0t, bb0, Wi1t, Wh1t, bb1)


def kernel(edge_index, x, edge_weight, W1, b1, eps1, W2, b2, eps2, W3, b3,
           eps3, W4, b4, eps4, W_ih0, W_hh0, b_ih0, b_hh0, W_ih1, W_hh1,
           b_ih1, b_hh1):
    layers = ((W1, b1, eps1), (W2, b2, eps2), (W3, b3, eps3), (W4, b4, eps4))
    src_arr = edge_index[0]
    dst_arr = edge_index[1]
    xs = []
    h = x
    for W, b, eps in layers:
        agg = _sc_segment_relu_sum(src_arr, dst_arr, h, edge_weight)
        h = _tc_linear(h, agg, W.T, b, eps)
        xs.append(h)
    bb0 = jnp.reshape(b_ih0 + b_hh0, (1, -1))
    bb1 = jnp.reshape(b_ih1 + b_hh1, (1, -1))
    return _tc_lstm(xs, W_ih0.T, W_hh0.T, bb0, W_ih1.T, W_hh1.T, bb1)
